# baseline (speedup 1.0000x reference)
import functools

import jax
import jax.numpy as jnp
from jax import lax
from jax.experimental import pallas as pl
from jax.experimental.pallas import tpu as pltpu
from jax.experimental.pallas import tpu_sc as plsc

N = 10000
E = 320000
D_IN = 128
H = 64
W = 128

NC = 2
NS = 16
NW = NC * NS
BATCH = 128
SLOW_CORE = 0
K_SLOW = 44
K_FAST = 113
CAP = NS * (K_SLOW + K_FAST) * BATCH
N_PAD = -(-(N + 1) // (NS * 8)) * (NS * 8)
RPT = N_PAD // NS


def _edge_agg_body(y_hbm, src_hbm, dst_hbm, agg_out, src_v, dst_v, b0, acc_sh, g0):
    c = lax.axis_index("c")
    s = lax.axis_index("s")
    wid = s * NC + c
    base = s * RPT

    zv = jnp.zeros((16,), jnp.float32)

    def zrow(r, _):
        def zcol(k, _):
            b0[r, pl.ds(k * 16, 16)] = zv
            return 0
        return lax.fori_loop(0, W // 16, zcol, 0)
    lax.fori_loop(0, BATCH, zrow, 0)

    nfull = RPT // BATCH
    rem = RPT % BATCH

    def zacc(i, _):
        pltpu.sync_copy(b0, acc_sh.at[pl.ds(base + i * BATCH, BATCH)])
        return 0
    lax.fori_loop(0, nfull, zacc, 0)
    if rem:
        pltpu.sync_copy(b0.at[pl.ds(0, rem)],
                        acc_sh.at[pl.ds(base + nfull * BATCH, rem)])

    pltpu.sync_copy(src_hbm.at[wid], src_v)
    pltpu.sync_copy(dst_hbm.at[wid], dst_v)

    plsc.subcore_barrier()

    def chunk(j, _):
        pltpu.async_copy(y_hbm.at[src_v.at[j]], b0, g0).wait()
        pltpu.sync_copy(b0, acc_sh.at[dst_v.at[j]], add=True)
        return 0
    lax.fori_loop(0, K_SLOW, chunk, 0)

    @pl.when(c != SLOW_CORE)
    def _extra():
        lax.fori_loop(K_SLOW, K_FAST, chunk, 0)

    plsc.subcore_barrier()

    def wout(i, _):
        pltpu.sync_copy(acc_sh.at[pl.ds(base + i * BATCH, BATCH)], b0)
        pltpu.sync_copy(b0, agg_out.at[c, pl.ds(base + i * BATCH, BATCH)])
        return 0
    lax.fori_loop(0, nfull, wout, 0)
    if rem:
        pltpu.sync_copy(acc_sh.at[pl.ds(base + nfull * BATCH, rem)],
                        b0.at[pl.ds(0, rem)])
        pltpu.sync_copy(b0.at[pl.ds(0, rem)],
                        agg_out.at[c, pl.ds(base + nfull * BATCH, rem)])


_edge_agg = functools.partial(
    pl.kernel,
    mesh=plsc.VectorSubcoreMesh(core_axis_name="c", subcore_axis_name="s"),
    out_type=jax.ShapeDtypeStruct((NC, N_PAD, W), jnp.float32),
    scratch_types=[
        pltpu.VMEM((K_FAST, BATCH), jnp.int32),
        pltpu.VMEM((K_FAST, BATCH), jnp.int32),
        pltpu.VMEM((BATCH, W), jnp.float32),
        pltpu.VMEM_SHARED((N_PAD, W), jnp.float32),
        pltpu.SemaphoreType.DMA,
    ],
)(_edge_agg_body)


def _msg_table(h, wl):
    y = jnp.dot(h, wl, preferred_element_type=jnp.float32)
    ones = jnp.ones((h.shape[0], 1), jnp.float32)
    zeros = jnp.zeros((h.shape[0], W - H - 1), jnp.float32)
    return jnp.concatenate([y, ones, zeros], axis=1)


def _tc_pre_body(x_ref, wl_ref, wr_ref, b_ref, y_ref, z_ref):
    x = x_ref[...]
    y_ref[...] = _msg_table(x, wl_ref[...])
    z_ref[...] = (jnp.dot(x, wr_ref[...], preferred_element_type=jnp.float32)
                  + b_ref[...])


def _mean_relu(aggp_ref, z_ref):
    agg = (aggp_ref[0] + aggp_ref[1])[:N]
    deg = agg[:, H]
    deginv = 1.0 / jnp.maximum(deg, 1.0)
    return jnp.maximum(agg[:, :H] * deginv[:, None] + z_ref[...], 0.0)


def _tc_mid_body(aggp_ref, z_ref, wl_ref, wr_ref, b_ref, y_ref, z2_ref):
    h1 = _mean_relu(aggp_ref, z_ref)
    y_ref[...] = _msg_table(h1, wl_ref[...])
    z2_ref[...] = (jnp.dot(h1, wr_ref[...], preferred_element_type=jnp.float32)
                   + b_ref[...])


def _tc_post_body(aggp_ref, z_ref, wc_ref, bc_ref, out_ref):
    h2 = _mean_relu(aggp_ref, z_ref)
    out_ref[...] = (jnp.dot(h2, wc_ref[...], preferred_element_type=jnp.float32)
                    + bc_ref[...])


_tc_pre = pl.pallas_call(
    _tc_pre_body,
    out_shape=[jax.ShapeDtypeStruct((N, W), jnp.float32),
               jax.ShapeDtypeStruct((N, H), jnp.float32)],
)

_tc_mid = pl.pallas_call(
    _tc_mid_body,
    out_shape=[jax.ShapeDtypeStruct((N, W), jnp.float32),
               jax.ShapeDtypeStruct((N, H), jnp.float32)],
)

_tc_post = pl.pallas_call(
    _tc_post_body,
    out_shape=jax.ShapeDtypeStruct((N, 1), jnp.float32),
)


def kernel(x, edge_index, W1l, W1r, b1, W2l, W2r, b2, Wc, bc):
    pad = CAP - E
    src_p = jnp.concatenate([edge_index[0], jnp.zeros((pad,), jnp.int32)])
    dst_p = jnp.concatenate([edge_index[1], jnp.full((pad,), N, jnp.int32)])
    fill_n = (K_FAST - K_SLOW) * BATCH
    sfill = jnp.zeros((fill_n,), jnp.int32)
    dfill = jnp.full((fill_n,), N, jnp.int32)
    sparts, dparts, off = [], [], 0
    for wid_ in range(NW):
        k = K_SLOW if (wid_ % NC) == SLOW_CORE else K_FAST
        n = k * BATCH
        sparts.append(lax.slice(src_p, (off,), (off + n,)))
        dparts.append(lax.slice(dst_p, (off,), (off + n,)))
        if k < K_FAST:
            sparts.append(sfill)
            dparts.append(dfill)
        off += n
    assert off == CAP
    src3 = jnp.concatenate(sparts).reshape(NW, K_FAST, BATCH)
    dst3 = jnp.concatenate(dparts).reshape(NW, K_FAST, BATCH)

    y1, z1 = _tc_pre(x, W1l.T, W1r.T, b1.reshape(1, H))
    aggp1 = _edge_agg(y1, src3, dst3)
    y2, z2 = _tc_mid(aggp1, z1, W2l.T, W2r.T, b2.reshape(1, H))
    aggp2 = _edge_agg(y2, src3, dst3)
    out = _tc_post(aggp2, z2, Wc.T, bc.reshape(1, 1))
    return out.reshape(N)


# speedup vs baseline: 1.0134x; 1.0134x over previous
import functools

import jax
import jax.numpy as jnp
from jax import lax
from jax.experimental import pallas as pl
from jax.experimental.pallas import tpu as pltpu
from jax.experimental.pallas import tpu_sc as plsc

N = 10000
E = 320000
D_IN = 128
H = 64
W = 128

NC = 2
NS = 16
NW = NC * NS
BATCH = 128
SLOW_CORE = 0
K_SLOW = 54
K_FAST = 103
CAP = NS * (K_SLOW + K_FAST) * BATCH
N_PAD = -(-(N + 1) // (NS * 8)) * (NS * 8)
RPT = N_PAD // NS


def _edge_agg_body(y_hbm, src_hbm, dst_hbm, agg_out, src_v, dst_v, b0, acc_sh, g0):
    c = lax.axis_index("c")
    s = lax.axis_index("s")
    wid = s * NC + c
    base = s * RPT

    zv = jnp.zeros((16,), jnp.float32)

    def zrow(r, _):
        def zcol(k, _):
            b0[r, pl.ds(k * 16, 16)] = zv
            return 0
        return lax.fori_loop(0, W // 16, zcol, 0)
    lax.fori_loop(0, BATCH, zrow, 0)

    nfull = RPT // BATCH
    rem = RPT % BATCH

    def zacc(i, _):
        pltpu.sync_copy(b0, acc_sh.at[pl.ds(base + i * BATCH, BATCH)])
        return 0
    lax.fori_loop(0, nfull, zacc, 0)
    if rem:
        pltpu.sync_copy(b0.at[pl.ds(0, rem)],
                        acc_sh.at[pl.ds(base + nfull * BATCH, rem)])

    pltpu.sync_copy(src_hbm.at[wid], src_v)
    pltpu.sync_copy(dst_hbm.at[wid], dst_v)

    plsc.subcore_barrier()

    def chunk(j, _):
        pltpu.async_copy(y_hbm.at[src_v.at[j]], b0, g0).wait()
        pltpu.sync_copy(b0, acc_sh.at[dst_v.at[j]], add=True)
        return 0
    lax.fori_loop(0, K_SLOW, chunk, 0)

    @pl.when(c != SLOW_CORE)
    def _extra():
        lax.fori_loop(K_SLOW, K_FAST, chunk, 0)

    plsc.subcore_barrier()

    def wout(i, _):
        pltpu.sync_copy(acc_sh.at[pl.ds(base + i * BATCH, BATCH)], b0)
        pltpu.sync_copy(b0, agg_out.at[c, pl.ds(base + i * BATCH, BATCH)])
        return 0
    lax.fori_loop(0, nfull, wout, 0)
    if rem:
        pltpu.sync_copy(acc_sh.at[pl.ds(base + nfull * BATCH, rem)],
                        b0.at[pl.ds(0, rem)])
        pltpu.sync_copy(b0.at[pl.ds(0, rem)],
                        agg_out.at[c, pl.ds(base + nfull * BATCH, rem)])


_edge_agg = functools.partial(
    pl.kernel,
    mesh=plsc.VectorSubcoreMesh(core_axis_name="c", subcore_axis_name="s"),
    out_type=jax.ShapeDtypeStruct((NC, N_PAD, W), jnp.float32),
    scratch_types=[
        pltpu.VMEM((K_FAST, BATCH), jnp.int32),
        pltpu.VMEM((K_FAST, BATCH), jnp.int32),
        pltpu.VMEM((BATCH, W), jnp.float32),
        pltpu.VMEM_SHARED((N_PAD, W), jnp.float32),
        pltpu.SemaphoreType.DMA,
    ],
)(_edge_agg_body)


def _msg_table(h, wl):
    y = jnp.dot(h, wl, preferred_element_type=jnp.float32)
    ones = jnp.ones((h.shape[0], 1), jnp.float32)
    zeros = jnp.zeros((h.shape[0], W - H - 1), jnp.float32)
    return jnp.concatenate([y, ones, zeros], axis=1)


def _tc_pre_body(x_ref, wl_ref, wr_ref, b_ref, y_ref, z_ref):
    x = x_ref[...]
    y_ref[...] = _msg_table(x, wl_ref[...])
    z_ref[...] = (jnp.dot(x, wr_ref[...], preferred_element_type=jnp.float32)
                  + b_ref[...])


def _mean_relu(aggp_ref, z_ref):
    agg = (aggp_ref[0] + aggp_ref[1])[:N]
    deg = agg[:, H]
    deginv = 1.0 / jnp.maximum(deg, 1.0)
    return jnp.maximum(agg[:, :H] * deginv[:, None] + z_ref[...], 0.0)


def _tc_mid_body(aggp_ref, z_ref, wl_ref, wr_ref, b_ref, y_ref, z2_ref):
    h1 = _mean_relu(aggp_ref, z_ref)
    y_ref[...] = _msg_table(h1, wl_ref[...])
    z2_ref[...] = (jnp.dot(h1, wr_ref[...], preferred_element_type=jnp.float32)
                   + b_ref[...])


def _tc_post_body(aggp_ref, z_ref, wc_ref, bc_ref, out_ref):
    h2 = _mean_relu(aggp_ref, z_ref)
    out_ref[...] = (jnp.dot(h2, wc_ref[...], preferred_element_type=jnp.float32)
                    + bc_ref[...])


_tc_pre = pl.pallas_call(
    _tc_pre_body,
    out_shape=[jax.ShapeDtypeStruct((N, W), jnp.float32),
               jax.ShapeDtypeStruct((N, H), jnp.float32)],
)

_tc_mid = pl.pallas_call(
    _tc_mid_body,
    out_shape=[jax.ShapeDtypeStruct((N, W), jnp.float32),
               jax.ShapeDtypeStruct((N, H), jnp.float32)],
)

_tc_post = pl.pallas_call(
    _tc_post_body,
    out_shape=jax.ShapeDtypeStruct((N, 1), jnp.float32),
)


def kernel(x, edge_index, W1l, W1r, b1, W2l, W2r, b2, Wc, bc):
    pad = CAP - E
    src_p = jnp.concatenate([edge_index[0], jnp.zeros((pad,), jnp.int32)])
    dst_p = jnp.concatenate([edge_index[1], jnp.full((pad,), N, jnp.int32)])
    fill_n = (K_FAST - K_SLOW) * BATCH
    sfill = jnp.zeros((fill_n,), jnp.int32)
    dfill = jnp.full((fill_n,), N, jnp.int32)
    sparts, dparts, off = [], [], 0
    for wid_ in range(NW):
        k = K_SLOW if (wid_ % NC) == SLOW_CORE else K_FAST
        n = k * BATCH
        sparts.append(lax.slice(src_p, (off,), (off + n,)))
        dparts.append(lax.slice(dst_p, (off,), (off + n,)))
        if k < K_FAST:
            sparts.append(sfill)
            dparts.append(dfill)
        off += n
    assert off == CAP
    src3 = jnp.concatenate(sparts).reshape(NW, K_FAST, BATCH)
    dst3 = jnp.concatenate(dparts).reshape(NW, K_FAST, BATCH)

    y1, z1 = _tc_pre(x, W1l.T, W1r.T, b1.reshape(1, H))
    aggp1 = _edge_agg(y1, src3, dst3)
    y2, z2 = _tc_mid(aggp1, z1, W2l.T, W2r.T, b2.reshape(1, H))
    aggp2 = _edge_agg(y2, src3, dst3)
    out = _tc_post(aggp2, z2, Wc.T, bc.reshape(1, 1))
    return out.reshape(N)


# speedup vs baseline: 1.1759x; 1.1603x over previous
import functools

import jax
import jax.numpy as jnp
from jax import lax
from jax.experimental import pallas as pl
from jax.experimental.pallas import tpu as pltpu
from jax.experimental.pallas import tpu_sc as plsc

N = 10000
E = 320000
D_IN = 128
H = 64
W = 128

NC = 2
NS = 16
NW = NC * NS
BATCH = 128
SLOW_CORE = 0
K_SLOW = 66
K_FAST = 91
CAP = NS * (K_SLOW + K_FAST) * BATCH
N_PAD = -(-(N + 1) // (NS * 8)) * (NS * 8)
RPT = N_PAD // NS


def _edge_agg_body(y_hbm, src_hbm, dst_hbm, agg_out, src_v, dst_v, b0, acc_sh, g0):
    c = lax.axis_index("c")
    s = lax.axis_index("s")
    wid = s * NC + c
    base = s * RPT

    zv = jnp.zeros((16,), jnp.float32)

    def zrow(r, _):
        def zcol(k, _):
            b0[r, pl.ds(k * 16, 16)] = zv
            return 0
        return lax.fori_loop(0, W // 16, zcol, 0)
    lax.fori_loop(0, BATCH, zrow, 0)

    nfull = RPT // BATCH
    rem = RPT % BATCH

    def zacc(i, _):
        pltpu.sync_copy(b0, acc_sh.at[pl.ds(base + i * BATCH, BATCH)])
        return 0
    lax.fori_loop(0, nfull, zacc, 0)
    if rem:
        pltpu.sync_copy(b0.at[pl.ds(0, rem)],
                        acc_sh.at[pl.ds(base + nfull * BATCH, rem)])

    pltpu.sync_copy(src_hbm.at[wid], src_v)
    pltpu.sync_copy(dst_hbm.at[wid], dst_v)

    plsc.subcore_barrier()

    def chunk(j, _):
        pltpu.async_copy(y_hbm.at[src_v.at[j]], b0, g0).wait()
        pltpu.sync_copy(b0, acc_sh.at[dst_v.at[j]], add=True)
        return 0
    lax.fori_loop(0, K_SLOW, chunk, 0)

    @pl.when(c != SLOW_CORE)
    def _extra():
        lax.fori_loop(K_SLOW, K_FAST, chunk, 0)

    plsc.subcore_barrier()

    def wout(i, _):
        pltpu.sync_copy(acc_sh.at[pl.ds(base + i * BATCH, BATCH)], b0)
        pltpu.sync_copy(b0, agg_out.at[c, pl.ds(base + i * BATCH, BATCH)])
        return 0
    lax.fori_loop(0, nfull, wout, 0)
    if rem:
        pltpu.sync_copy(acc_sh.at[pl.ds(base + nfull * BATCH, rem)],
                        b0.at[pl.ds(0, rem)])
        pltpu.sync_copy(b0.at[pl.ds(0, rem)],
                        agg_out.at[c, pl.ds(base + nfull * BATCH, rem)])


_edge_agg = functools.partial(
    pl.kernel,
    mesh=plsc.VectorSubcoreMesh(core_axis_name="c", subcore_axis_name="s"),
    out_type=jax.ShapeDtypeStruct((NC, N_PAD, W), jnp.float32),
    scratch_types=[
        pltpu.VMEM((K_FAST, BATCH), jnp.int32),
        pltpu.VMEM((K_FAST, BATCH), jnp.int32),
        pltpu.VMEM((BATCH, W), jnp.float32),
        pltpu.VMEM_SHARED((N_PAD, W), jnp.float32),
        pltpu.SemaphoreType.DMA,
    ],
)(_edge_agg_body)


def _msg_table(h, wl):
    y = jnp.dot(h, wl, preferred_element_type=jnp.float32)
    ones = jnp.ones((h.shape[0], 1), jnp.float32)
    zeros = jnp.zeros((h.shape[0], W - H - 1), jnp.float32)
    return jnp.concatenate([y, ones, zeros], axis=1)


def _tc_pre_body(x_ref, wl_ref, wr_ref, b_ref, y_ref, z_ref):
    x = x_ref[...]
    y_ref[...] = _msg_table(x, wl_ref[...])
    z_ref[...] = (jnp.dot(x, wr_ref[...], preferred_element_type=jnp.float32)
                  + b_ref[...])


def _mean_relu(aggp_ref, z_ref):
    agg = (aggp_ref[0] + aggp_ref[1])[:N]
    deg = agg[:, H]
    deginv = 1.0 / jnp.maximum(deg, 1.0)
    return jnp.maximum(agg[:, :H] * deginv[:, None] + z_ref[...], 0.0)


def _tc_mid_body(aggp_ref, z_ref, wl_ref, wr_ref, b_ref, y_ref, z2_ref):
    h1 = _mean_relu(aggp_ref, z_ref)
    y_ref[...] = _msg_table(h1, wl_ref[...])
    z2_ref[...] = (jnp.dot(h1, wr_ref[...], preferred_element_type=jnp.float32)
                   + b_ref[...])


def _tc_post_body(aggp_ref, z_ref, wc_ref, bc_ref, out_ref):
    h2 = _mean_relu(aggp_ref, z_ref)
    out_ref[...] = (jnp.dot(h2, wc_ref[...], preferred_element_type=jnp.float32)
                    + bc_ref[...])


_tc_pre = pl.pallas_call(
    _tc_pre_body,
    out_shape=[jax.ShapeDtypeStruct((N, W), jnp.float32),
               jax.ShapeDtypeStruct((N, H), jnp.float32)],
)

_tc_mid = pl.pallas_call(
    _tc_mid_body,
    out_shape=[jax.ShapeDtypeStruct((N, W), jnp.float32),
               jax.ShapeDtypeStruct((N, H), jnp.float32)],
)

_tc_post = pl.pallas_call(
    _tc_post_body,
    out_shape=jax.ShapeDtypeStruct((N, 1), jnp.float32),
)


def kernel(x, edge_index, W1l, W1r, b1, W2l, W2r, b2, Wc, bc):
    pad = CAP - E
    src_p = jnp.concatenate([edge_index[0], jnp.zeros((pad,), jnp.int32)])
    dst_p = jnp.concatenate([edge_index[1], jnp.full((pad,), N, jnp.int32)])
    fill_n = (K_FAST - K_SLOW) * BATCH
    sfill = jnp.zeros((fill_n,), jnp.int32)
    dfill = jnp.full((fill_n,), N, jnp.int32)
    sparts, dparts, off = [], [], 0
    for wid_ in range(NW):
        k = K_SLOW if (wid_ % NC) == SLOW_CORE else K_FAST
        n = k * BATCH
        sparts.append(lax.slice(src_p, (off,), (off + n,)))
        dparts.append(lax.slice(dst_p, (off,), (off + n,)))
        if k < K_FAST:
            sparts.append(sfill)
            dparts.append(dfill)
        off += n
    assert off == CAP
    src3 = jnp.concatenate(sparts).reshape(NW, K_FAST, BATCH)
    dst3 = jnp.concatenate(dparts).reshape(NW, K_FAST, BATCH)

    y1, z1 = _tc_pre(x, W1l.T, W1r.T, b1.reshape(1, H))
    aggp1 = _edge_agg(y1, src3, dst3)
    y2, z2 = _tc_mid(aggp1, z1, W2l.T, W2r.T, b2.reshape(1, H))
    aggp2 = _edge_agg(y2, src3, dst3)
    out = _tc_post(aggp2, z2, Wc.T, bc.reshape(1, 1))
    return out.reshape(N)


# speedup vs baseline: 1.2471x; 1.0605x over previous
import functools

import jax
import jax.numpy as jnp
from jax import lax
from jax.experimental import pallas as pl
from jax.experimental.pallas import tpu as pltpu
from jax.experimental.pallas import tpu_sc as plsc

N = 10000
E = 320000
D_IN = 128
H = 64
W = 128

NC = 2
NS = 16
NW = NC * NS
BATCH = 128
SLOW_CORE = 0
K_SLOW = 72
K_FAST = 85
CAP = NS * (K_SLOW + K_FAST) * BATCH
N_PAD = -(-(N + 1) // (NS * 8)) * (NS * 8)
RPT = N_PAD // NS


def _edge_agg_body(y_hbm, src_hbm, dst_hbm, agg_out, src_v, dst_v, b0, acc_sh, g0):
    c = lax.axis_index("c")
    s = lax.axis_index("s")
    wid = s * NC + c
    base = s * RPT

    zv = jnp.zeros((16,), jnp.float32)

    def zrow(r, _):
        def zcol(k, _):
            b0[r, pl.ds(k * 16, 16)] = zv
            return 0
        return lax.fori_loop(0, W // 16, zcol, 0)
    lax.fori_loop(0, BATCH, zrow, 0)

    nfull = RPT // BATCH
    rem = RPT % BATCH

    def zacc(i, _):
        pltpu.sync_copy(b0, acc_sh.at[pl.ds(base + i * BATCH, BATCH)])
        return 0
    lax.fori_loop(0, nfull, zacc, 0)
    if rem:
        pltpu.sync_copy(b0.at[pl.ds(0, rem)],
                        acc_sh.at[pl.ds(base + nfull * BATCH, rem)])

    pltpu.sync_copy(src_hbm.at[wid], src_v)
    pltpu.sync_copy(dst_hbm.at[wid], dst_v)

    plsc.subcore_barrier()

    def chunk(j, _):
        pltpu.async_copy(y_hbm.at[src_v.at[j]], b0, g0).wait()
        pltpu.sync_copy(b0, acc_sh.at[dst_v.at[j]], add=True)
        return 0
    lax.fori_loop(0, K_SLOW, chunk, 0)

    @pl.when(c != SLOW_CORE)
    def _extra():
        lax.fori_loop(K_SLOW, K_FAST, chunk, 0)

    plsc.subcore_barrier()

    def wout(i, _):
        pltpu.sync_copy(acc_sh.at[pl.ds(base + i * BATCH, BATCH)], b0)
        pltpu.sync_copy(b0, agg_out.at[c, pl.ds(base + i * BATCH, BATCH)])
        return 0
    lax.fori_loop(0, nfull, wout, 0)
    if rem:
        pltpu.sync_copy(acc_sh.at[pl.ds(base + nfull * BATCH, rem)],
                        b0.at[pl.ds(0, rem)])
        pltpu.sync_copy(b0.at[pl.ds(0, rem)],
                        agg_out.at[c, pl.ds(base + nfull * BATCH, rem)])


_edge_agg = functools.partial(
    pl.kernel,
    mesh=plsc.VectorSubcoreMesh(core_axis_name="c", subcore_axis_name="s"),
    out_type=jax.ShapeDtypeStruct((NC, N_PAD, W), jnp.float32),
    scratch_types=[
        pltpu.VMEM((K_FAST, BATCH), jnp.int32),
        pltpu.VMEM((K_FAST, BATCH), jnp.int32),
        pltpu.VMEM((BATCH, W), jnp.float32),
        pltpu.VMEM_SHARED((N_PAD, W), jnp.float32),
        pltpu.SemaphoreType.DMA,
    ],
)(_edge_agg_body)


def _msg_table(h, wl):
    y = jnp.dot(h, wl, preferred_element_type=jnp.float32)
    ones = jnp.ones((h.shape[0], 1), jnp.float32)
    zeros = jnp.zeros((h.shape[0], W - H - 1), jnp.float32)
    return jnp.concatenate([y, ones, zeros], axis=1)


def _tc_pre_body(x_ref, wl_ref, wr_ref, b_ref, y_ref, z_ref):
    x = x_ref[...]
    y_ref[...] = _msg_table(x, wl_ref[...])
    z_ref[...] = (jnp.dot(x, wr_ref[...], preferred_element_type=jnp.float32)
                  + b_ref[...])


def _mean_relu(aggp_ref, z_ref):
    agg = (aggp_ref[0] + aggp_ref[1])[:N]
    deg = agg[:, H]
    deginv = 1.0 / jnp.maximum(deg, 1.0)
    return jnp.maximum(agg[:, :H] * deginv[:, None] + z_ref[...], 0.0)


def _tc_mid_body(aggp_ref, z_ref, wl_ref, wr_ref, b_ref, y_ref, z2_ref):
    h1 = _mean_relu(aggp_ref, z_ref)
    y_ref[...] = _msg_table(h1, wl_ref[...])
    z2_ref[...] = (jnp.dot(h1, wr_ref[...], preferred_element_type=jnp.float32)
                   + b_ref[...])


def _tc_post_body(aggp_ref, z_ref, wc_ref, bc_ref, out_ref):
    h2 = _mean_relu(aggp_ref, z_ref)
    out_ref[...] = (jnp.dot(h2, wc_ref[...], preferred_element_type=jnp.float32)
                    + bc_ref[...])


_tc_pre = pl.pallas_call(
    _tc_pre_body,
    out_shape=[jax.ShapeDtypeStruct((N, W), jnp.float32),
               jax.ShapeDtypeStruct((N, H), jnp.float32)],
)

_tc_mid = pl.pallas_call(
    _tc_mid_body,
    out_shape=[jax.ShapeDtypeStruct((N, W), jnp.float32),
               jax.ShapeDtypeStruct((N, H), jnp.float32)],
)

_tc_post = pl.pallas_call(
    _tc_post_body,
    out_shape=jax.ShapeDtypeStruct((N, 1), jnp.float32),
)


def kernel(x, edge_index, W1l, W1r, b1, W2l, W2r, b2, Wc, bc):
    pad = CAP - E
    src_p = jnp.concatenate([edge_index[0], jnp.zeros((pad,), jnp.int32)])
    dst_p = jnp.concatenate([edge_index[1], jnp.full((pad,), N, jnp.int32)])
    fill_n = (K_FAST - K_SLOW) * BATCH
    sfill = jnp.zeros((fill_n,), jnp.int32)
    dfill = jnp.full((fill_n,), N, jnp.int32)
    sparts, dparts, off = [], [], 0
    for wid_ in range(NW):
        k = K_SLOW if (wid_ % NC) == SLOW_CORE else K_FAST
        n = k * BATCH
        sparts.append(lax.slice(src_p, (off,), (off + n,)))
        dparts.append(lax.slice(dst_p, (off,), (off + n,)))
        if k < K_FAST:
            sparts.append(sfill)
            dparts.append(dfill)
        off += n
    assert off == CAP
    src3 = jnp.concatenate(sparts).reshape(NW, K_FAST, BATCH)
    dst3 = jnp.concatenate(dparts).reshape(NW, K_FAST, BATCH)

    y1, z1 = _tc_pre(x, W1l.T, W1r.T, b1.reshape(1, H))
    aggp1 = _edge_agg(y1, src3, dst3)
    y2, z2 = _tc_mid(aggp1, z1, W2l.T, W2r.T, b2.reshape(1, H))
    aggp2 = _edge_agg(y2, src3, dst3)
    out = _tc_post(aggp2, z2, Wc.T, bc.reshape(1, 1))
    return out.reshape(N)


# speedup vs baseline: 1.3210x; 1.0592x over previous
import functools

import jax
import jax.numpy as jnp
from jax import lax
from jax.experimental import pallas as pl
from jax.experimental.pallas import tpu as pltpu
from jax.experimental.pallas import tpu_sc as plsc

N = 10000
E = 320000
D_IN = 128
H = 64
W = 128

NC = 2
NS = 16
NW = NC * NS
BATCH = 128
SLOW_CORE = 0
K_SLOW = 78
K_FAST = 79
CAP = NS * (K_SLOW + K_FAST) * BATCH
N_PAD = -(-(N + 1) // (NS * 8)) * (NS * 8)
RPT = N_PAD // NS


def _edge_agg_body(y_hbm, src_hbm, dst_hbm, agg_out, src_v, dst_v, b0, acc_sh, g0):
    c = lax.axis_index("c")
    s = lax.axis_index("s")
    wid = s * NC + c
    base = s * RPT

    zv = jnp.zeros((16,), jnp.float32)

    def zrow(r, _):
        def zcol(k, _):
            b0[r, pl.ds(k * 16, 16)] = zv
            return 0
        return lax.fori_loop(0, W // 16, zcol, 0)
    lax.fori_loop(0, BATCH, zrow, 0)

    nfull = RPT // BATCH
    rem = RPT % BATCH

    def zacc(i, _):
        pltpu.sync_copy(b0, acc_sh.at[pl.ds(base + i * BATCH, BATCH)])
        return 0
    lax.fori_loop(0, nfull, zacc, 0)
    if rem:
        pltpu.sync_copy(b0.at[pl.ds(0, rem)],
                        acc_sh.at[pl.ds(base + nfull * BATCH, rem)])

    pltpu.sync_copy(src_hbm.at[wid], src_v)
    pltpu.sync_copy(dst_hbm.at[wid], dst_v)

    plsc.subcore_barrier()

    def chunk(j, _):
        pltpu.async_copy(y_hbm.at[src_v.at[j]], b0, g0).wait()
        pltpu.sync_copy(b0, acc_sh.at[dst_v.at[j]], add=True)
        return 0
    lax.fori_loop(0, K_SLOW, chunk, 0)

    @pl.when(c != SLOW_CORE)
    def _extra():
        lax.fori_loop(K_SLOW, K_FAST, chunk, 0)

    plsc.subcore_barrier()

    def wout(i, _):
        pltpu.sync_copy(acc_sh.at[pl.ds(base + i * BATCH, BATCH)], b0)
        pltpu.sync_copy(b0, agg_out.at[c, pl.ds(base + i * BATCH, BATCH)])
        return 0
    lax.fori_loop(0, nfull, wout, 0)
    if rem:
        pltpu.sync_copy(acc_sh.at[pl.ds(base + nfull * BATCH, rem)],
                        b0.at[pl.ds(0, rem)])
        pltpu.sync_copy(b0.at[pl.ds(0, rem)],
                        agg_out.at[c, pl.ds(base + nfull * BATCH, rem)])


_edge_agg = functools.partial(
    pl.kernel,
    mesh=plsc.VectorSubcoreMesh(core_axis_name="c", subcore_axis_name="s"),
    out_type=jax.ShapeDtypeStruct((NC, N_PAD, W), jnp.float32),
    scratch_types=[
        pltpu.VMEM((K_FAST, BATCH), jnp.int32),
        pltpu.VMEM((K_FAST, BATCH), jnp.int32),
        pltpu.VMEM((BATCH, W), jnp.float32),
        pltpu.VMEM_SHARED((N_PAD, W), jnp.float32),
        pltpu.SemaphoreType.DMA,
    ],
)(_edge_agg_body)


def _msg_table(h, wl):
    y = jnp.dot(h, wl, preferred_element_type=jnp.float32)
    ones = jnp.ones((h.shape[0], 1), jnp.float32)
    zeros = jnp.zeros((h.shape[0], W - H - 1), jnp.float32)
    return jnp.concatenate([y, ones, zeros], axis=1)


def _tc_pre_body(x_ref, wl_ref, wr_ref, b_ref, y_ref, z_ref):
    x = x_ref[...]
    y_ref[...] = _msg_table(x, wl_ref[...])
    z_ref[...] = (jnp.dot(x, wr_ref[...], preferred_element_type=jnp.float32)
                  + b_ref[...])


def _mean_relu(aggp_ref, z_ref):
    agg = (aggp_ref[0] + aggp_ref[1])[:N]
    deg = agg[:, H]
    deginv = 1.0 / jnp.maximum(deg, 1.0)
    return jnp.maximum(agg[:, :H] * deginv[:, None] + z_ref[...], 0.0)


def _tc_mid_body(aggp_ref, z_ref, wl_ref, wr_ref, b_ref, y_ref, z2_ref):
    h1 = _mean_relu(aggp_ref, z_ref)
    y_ref[...] = _msg_table(h1, wl_ref[...])
    z2_ref[...] = (jnp.dot(h1, wr_ref[...], preferred_element_type=jnp.float32)
                   + b_ref[...])


def _tc_post_body(aggp_ref, z_ref, wc_ref, bc_ref, out_ref):
    h2 = _mean_relu(aggp_ref, z_ref)
    out_ref[...] = (jnp.dot(h2, wc_ref[...], preferred_element_type=jnp.float32)
                    + bc_ref[...])


_tc_pre = pl.pallas_call(
    _tc_pre_body,
    out_shape=[jax.ShapeDtypeStruct((N, W), jnp.float32),
               jax.ShapeDtypeStruct((N, H), jnp.float32)],
)

_tc_mid = pl.pallas_call(
    _tc_mid_body,
    out_shape=[jax.ShapeDtypeStruct((N, W), jnp.float32),
               jax.ShapeDtypeStruct((N, H), jnp.float32)],
)

_tc_post = pl.pallas_call(
    _tc_post_body,
    out_shape=jax.ShapeDtypeStruct((N, 1), jnp.float32),
)


def kernel(x, edge_index, W1l, W1r, b1, W2l, W2r, b2, Wc, bc):
    pad = CAP - E
    src_p = jnp.concatenate([edge_index[0], jnp.zeros((pad,), jnp.int32)])
    dst_p = jnp.concatenate([edge_index[1], jnp.full((pad,), N, jnp.int32)])
    fill_n = (K_FAST - K_SLOW) * BATCH
    sfill = jnp.zeros((fill_n,), jnp.int32)
    dfill = jnp.full((fill_n,), N, jnp.int32)
    sparts, dparts, off = [], [], 0
    for wid_ in range(NW):
        k = K_SLOW if (wid_ % NC) == SLOW_CORE else K_FAST
        n = k * BATCH
        sparts.append(lax.slice(src_p, (off,), (off + n,)))
        dparts.append(lax.slice(dst_p, (off,), (off + n,)))
        if k < K_FAST:
            sparts.append(sfill)
            dparts.append(dfill)
        off += n
    assert off == CAP
    src3 = jnp.concatenate(sparts).reshape(NW, K_FAST, BATCH)
    dst3 = jnp.concatenate(dparts).reshape(NW, K_FAST, BATCH)

    y1, z1 = _tc_pre(x, W1l.T, W1r.T, b1.reshape(1, H))
    aggp1 = _edge_agg(y1, src3, dst3)
    y2, z2 = _tc_mid(aggp1, z1, W2l.T, W2r.T, b2.reshape(1, H))
    aggp2 = _edge_agg(y2, src3, dst3)
    out = _tc_post(aggp2, z2, Wc.T, bc.reshape(1, 1))
    return out.reshape(N)
